# baseline (speedup 1.0000x reference)
import jax
import jax.numpy as jnp
import numpy as np
from jax import lax
from jax.experimental import pallas as pl
from jax.experimental.pallas import tpu as pltpu
from jax.experimental.pallas import tpu_sc as plsc

N = 4194304
NUM_FEATURES = 65536
NC = 2
NS = 16
NW = NC * NS
PER_W = N // NW
C = 8192
CHUNKS = PER_W // C
G = 128
NG = C // G
LANES = 16

MAGIC = np.float32(8388608.0)
SCALE = np.float32(10.0 ** 7)
GRIDF = np.float32(256.0)
X0 = np.float32(-10.0)
Y0 = np.float32(-5.0)
WE = np.float32(20.0)
HE = np.float32(10.0)


def _bin(c, origin, extent):
    v = c * SCALE
    a = jnp.abs(v)
    r = (a + MAGIC) - MAGIC
    r = jnp.where(a < MAGIC, r, a)
    r = jnp.where(v < np.float32(0.0), -r, r)
    c7 = r / SCALE
    t = (c7 - origin) / extent * GRIDF
    ti = t.astype(jnp.int32)
    return jnp.clip(ti, 0, 255)


def _body(cx_hbm, cy_hbm, x_hbm, w_hbm, b_hbm, out_hbm,
          cx_v, cy_v, x_v, idx_v, w_v, b_v, out_v, sem_in, sem_g, sem_o):
    wid = lax.axis_index("s") * NC + lax.axis_index("c")
    base = wid * PER_W

    def compute_idx(k, p):
        off = base + k * C
        h0 = pltpu.async_copy(cx_hbm.at[pl.ds(off, C)], cx_v, sem_in)
        h1 = pltpu.async_copy(cy_hbm.at[pl.ds(off, C)], cy_v, sem_in)
        hx = pltpu.async_copy(x_hbm.at[pl.ds(off, C)], x_v.at[p], sem_in)
        h0.wait()
        h1.wait()
        hx.wait()

        def idx_body(j, carry2):
            sl = pl.ds(j * LANES, LANES)
            xi = _bin(cx_v[sl], X0, WE)
            yi = _bin(cy_v[sl], Y0, HE)
            idx_v[p, sl] = yi * 256 + xi
            return carry2

        lax.fori_loop(0, C // LANES, idx_body, 0)
        hw = pltpu.async_copy(w_hbm.at[idx_v.at[p]], w_v.at[p], sem_g)
        hb = pltpu.async_copy(b_hbm.at[idx_v.at[p]], b_v.at[p], sem_g)
        return hw, hb

    def finish(k, p, hw, hb, prev_out):
        hw.wait()
        hb.wait()
        if prev_out is not None:
            prev_out.wait()

        def fma_body(j, carry2):
            sl = pl.ds(j * LANES, LANES)
            out_v[p, sl] = w_v[p, sl] * x_v[p, sl] + b_v[p, sl]
            return carry2

        lax.fori_loop(0, C // LANES, fma_body, 0)
        off = base + k * C
        return pltpu.async_copy(out_v.at[p], out_hbm.at[pl.ds(off, C)], sem_o)

    pending = None
    prev_out = [None, None]
    for k in range(CHUNKS):
        p = k % 2
        h = compute_idx(k, p)
        if pending is not None:
            pk, pp, phw, phb = pending
            prev_out[pp] = finish(pk, pp, phw, phb, prev_out[pp])
        pending = (k, p, h[0], h[1])
    pk, pp, phw, phb = pending
    prev_out[pp] = finish(pk, pp, phw, phb, prev_out[pp])
    for p in range(2):
        if prev_out[p] is not None:
            prev_out[p].wait()


def kernel(input, x, weight, bias):
    run = pl.kernel(
        _body,
        out_type=jax.ShapeDtypeStruct((N,), jnp.float32),
        mesh=plsc.VectorSubcoreMesh(core_axis_name="c", subcore_axis_name="s"),
        compiler_params=pltpu.CompilerParams(
            needs_layout_passes=False, use_tc_tiling_on_sc=False),
        scratch_types=[
            pltpu.VMEM((C,), jnp.float32),
            pltpu.VMEM((C,), jnp.float32),
            pltpu.VMEM((2, C), jnp.float32),
            pltpu.VMEM((2, C), jnp.int32),
            pltpu.VMEM((2, C), jnp.float32),
            pltpu.VMEM((2, C), jnp.float32),
            pltpu.VMEM((2, C), jnp.float32),
            pltpu.SemaphoreType.DMA,
            pltpu.SemaphoreType.DMA,
            pltpu.SemaphoreType.DMA,
        ],
    )
    return run(input[:, 0], input[:, 1], x, weight, bias)
